# Initial kernel scaffold; baseline (speedup 1.0000x reference)
import jax
import jax.numpy as jnp
from jax.experimental import pallas as pl


def kernel(embeddings, edge_index, node_w, node_b, edge_w, edge_b):
    raise NotImplementedError("write your pallas kernel here")


# speedup vs baseline: 6.2138x; 6.2138x over previous
import functools

import jax
import jax.numpy as jnp
from jax import lax
from jax.experimental import pallas as pl
from jax.experimental.pallas import tpu as pltpu
from jax.experimental.pallas import tpu_sc as plsc

HIDDEN = 128
N_NODE_FEAT = 128
N_EDGE_FEAT = 16
N_NODES = 10000
N_EDGES = 320000


_ROWS_BLK = 1000


def _p_tables_body(x_ref, ws_ref, wd_ref, eb_ref, ps_ref, pd_ref):
    x = x_ref[...]
    ps_ref[...] = (
        jnp.dot(x, ws_ref[...], preferred_element_type=jnp.float32,
                precision=lax.Precision.HIGHEST)
        + eb_ref[...]
    )
    pd_ref[...] = jnp.dot(x, wd_ref[...], preferred_element_type=jnp.float32,
                          precision=lax.Precision.HIGHEST)


def _p_tables(emb, ws, wd, eb):
    grid = (N_NODES // _ROWS_BLK,)
    return pl.pallas_call(
        _p_tables_body,
        grid=grid,
        in_specs=[
            pl.BlockSpec((_ROWS_BLK, HIDDEN), lambda i: (i, 0)),
            pl.BlockSpec((HIDDEN, N_EDGE_FEAT), lambda i: (0, 0)),
            pl.BlockSpec((HIDDEN, N_EDGE_FEAT), lambda i: (0, 0)),
            pl.BlockSpec((1, N_EDGE_FEAT), lambda i: (0, 0)),
        ],
        out_specs=[
            pl.BlockSpec((_ROWS_BLK, N_EDGE_FEAT), lambda i: (i, 0)),
            pl.BlockSpec((_ROWS_BLK, N_EDGE_FEAT), lambda i: (i, 0)),
        ],
        out_shape=[
            jax.ShapeDtypeStruct((N_NODES, N_EDGE_FEAT), jnp.float32),
            jax.ShapeDtypeStruct((N_NODES, N_EDGE_FEAT), jnp.float32),
        ],
    )(emb, ws, wd, eb)


def _node_body(x_ref, w_ref, b_ref, o_ref):
    o_ref[...] = (
        jnp.dot(x_ref[...], w_ref[...], preferred_element_type=jnp.float32,
                precision=lax.Precision.HIGHEST)
        + b_ref[...]
    )


def _node_linear(emb, w, b):
    grid = (N_NODES // _ROWS_BLK,)
    return pl.pallas_call(
        _node_body,
        grid=grid,
        in_specs=[
            pl.BlockSpec((_ROWS_BLK, HIDDEN), lambda i: (i, 0)),
            pl.BlockSpec((HIDDEN, N_NODE_FEAT), lambda i: (0, 0)),
            pl.BlockSpec((1, N_NODE_FEAT), lambda i: (0, 0)),
        ],
        out_specs=pl.BlockSpec((_ROWS_BLK, N_NODE_FEAT), lambda i: (i, 0)),
        out_shape=jax.ShapeDtypeStruct((N_NODES, N_NODE_FEAT), jnp.float32),
    )(emb, w, b)


_NC = 2
_NS = 16
_NW = _NC * _NS
_E_W = N_EDGES // _NW
_CHUNK = 128
_NFULL = _E_W // _CHUNK
_TAIL = _E_W - _NFULL * _CHUNK


def _edge_body(psrc_hbm, pdst_hbm, src_hbm, dst_hbm, out_hbm,
               src_v, dst_v,
               a0, a1, b0, b1, o0, o1,
               gs0, gs1, os0, os1):
    wid = lax.axis_index("s") * _NC + lax.axis_index("c")
    base = pl.multiple_of(wid * _E_W, 8)

    pltpu.sync_copy(src_hbm.at[pl.ds(base, _E_W)], src_v)
    pltpu.sync_copy(dst_hbm.at[pl.ds(base, _E_W)], dst_v)

    abufs = (a0, a1)
    bbufs = (b0, b1)
    obufs = (o0, o1)
    gsems = (gs0, gs1)
    osems = (os0, os1)

    def issue_gather(cc, k):
        off = pl.multiple_of(cc * _CHUNK, 8)
        pltpu.async_copy(psrc_hbm.at[src_v.at[pl.ds(off, _CHUNK)]],
                         abufs[k], gsems[k])
        pltpu.async_copy(pdst_hbm.at[dst_v.at[pl.ds(off, _CHUNK)]],
                         bbufs[k], gsems[k])

    def wait_gather(k):
        pltpu.make_async_copy(psrc_hbm.at[pl.ds(0, _CHUNK)], abufs[k],
                              gsems[k]).wait()
        pltpu.make_async_copy(pdst_hbm.at[pl.ds(0, _CHUNK)], bbufs[k],
                              gsems[k]).wait()

    def compute(k):
        a_ref, b_ref, o_ref = abufs[k], bbufs[k], obufs[k]

        @pl.loop(0, _CHUNK, unroll=8)
        def _(i):
            o_ref[i] = a_ref[i] + b_ref[i]

    def issue_out(cc, k):
        off = pl.multiple_of(base + cc * _CHUNK, 8)
        pltpu.async_copy(obufs[k], out_hbm.at[pl.ds(off, _CHUNK)], osems[k])

    def wait_out(k):
        pltpu.make_async_copy(obufs[k], out_hbm.at[pl.ds(base, _CHUNK)],
                              osems[k]).wait()

    issue_gather(0, 0)
    issue_gather(1, 1)

    for cc in (0, 1):
        k = cc % 2
        wait_gather(k)
        compute(k)
        issue_out(cc, k)
        issue_gather(cc + 2, k)

    @pl.loop(2, _NFULL - 2, step=2)
    def _(c):
        for k in range(2):
            cc = c + k
            wait_gather(k)
            wait_out(k)
            compute(k)
            issue_out(cc, k)
            issue_gather(cc + 2, k)

    for cc in (_NFULL - 2, _NFULL - 1):
        k = cc % 2
        wait_gather(k)
        wait_out(k)
        compute(k)
        issue_out(cc, k)

    t_off = pl.multiple_of(_NFULL * _CHUNK, 8)
    ta = abufs[0].at[pl.ds(0, _TAIL)]
    tb = bbufs[0].at[pl.ds(0, _TAIL)]
    to = obufs[0].at[pl.ds(0, _TAIL)]
    pltpu.async_copy(psrc_hbm.at[src_v.at[pl.ds(t_off, _TAIL)]], ta, gs0)
    pltpu.async_copy(pdst_hbm.at[dst_v.at[pl.ds(t_off, _TAIL)]], tb, gs0)
    wait_out(0)
    pltpu.make_async_copy(psrc_hbm.at[pl.ds(0, _TAIL)], ta, gs0).wait()
    pltpu.make_async_copy(pdst_hbm.at[pl.ds(0, _TAIL)], tb, gs0).wait()
    for i in range(_TAIL):
        to[i] = ta[i] + tb[i]
    pltpu.async_copy(to, out_hbm.at[pl.ds(base + t_off, _TAIL)], os0)

    pltpu.make_async_copy(to, out_hbm.at[pl.ds(base, _TAIL)], os0).wait()
    wait_out(1)


def _edge_decode(psrc, pdst, src, dst):
    mesh = plsc.VectorSubcoreMesh(core_axis_name="c", subcore_axis_name="s")
    f32 = jnp.float32
    run = pl.kernel(
        _edge_body,
        out_type=jax.ShapeDtypeStruct((N_EDGES, N_EDGE_FEAT), f32),
        mesh=mesh,
        compiler_params=pltpu.CompilerParams(use_tc_tiling_on_sc=False),
        scratch_types=[
            pltpu.VMEM((_E_W,), jnp.int32),
            pltpu.VMEM((_E_W,), jnp.int32),
            pltpu.VMEM((_CHUNK, N_EDGE_FEAT), f32),
            pltpu.VMEM((_CHUNK, N_EDGE_FEAT), f32),
            pltpu.VMEM((_CHUNK, N_EDGE_FEAT), f32),
            pltpu.VMEM((_CHUNK, N_EDGE_FEAT), f32),
            pltpu.VMEM((_CHUNK, N_EDGE_FEAT), f32),
            pltpu.VMEM((_CHUNK, N_EDGE_FEAT), f32),
            pltpu.SemaphoreType.DMA,
            pltpu.SemaphoreType.DMA,
            pltpu.SemaphoreType.DMA,
            pltpu.SemaphoreType.DMA,
        ],
    )
    return run(psrc, pdst, src, dst)


def kernel(embeddings, edge_index, node_w, node_b, edge_w, edge_b):
    src = edge_index[0].astype(jnp.int32)
    dst = edge_index[1].astype(jnp.int32)

    ws = edge_w[:, :HIDDEN].T
    wd = edge_w[:, HIDDEN:].T
    eb = edge_b.reshape(1, N_EDGE_FEAT)

    psrc, pdst = _p_tables(embeddings, ws, wd, eb)
    edge_hat = _edge_decode(psrc, pdst, src, dst)
    node_hat = _node_linear(embeddings, node_w.T, node_b.reshape(1, N_NODE_FEAT))
    return (node_hat, edge_hat)


# speedup vs baseline: 6.2786x; 1.0104x over previous
import functools

import jax
import jax.numpy as jnp
from jax import lax
from jax.experimental import pallas as pl
from jax.experimental.pallas import tpu as pltpu
from jax.experimental.pallas import tpu_sc as plsc

HIDDEN = 128
N_NODE_FEAT = 128
N_EDGE_FEAT = 16
N_NODES = 10000
N_EDGES = 320000


_ROWS_BLK = 1000


def _p_tables_body(x_ref, ws_ref, wd_ref, eb_ref, ps_ref, pd_ref):
    x = x_ref[...]
    ps_ref[...] = (
        jnp.dot(x, ws_ref[...], preferred_element_type=jnp.float32,
                precision=lax.Precision.HIGHEST)
        + eb_ref[...]
    )
    pd_ref[...] = jnp.dot(x, wd_ref[...], preferred_element_type=jnp.float32,
                          precision=lax.Precision.HIGHEST)


def _p_tables(emb, ws, wd, eb):
    grid = (N_NODES // _ROWS_BLK,)
    return pl.pallas_call(
        _p_tables_body,
        grid=grid,
        in_specs=[
            pl.BlockSpec((_ROWS_BLK, HIDDEN), lambda i: (i, 0)),
            pl.BlockSpec((HIDDEN, HIDDEN), lambda i: (0, 0)),
            pl.BlockSpec((HIDDEN, HIDDEN), lambda i: (0, 0)),
            pl.BlockSpec((1, HIDDEN), lambda i: (0, 0)),
        ],
        out_specs=[
            pl.BlockSpec((_ROWS_BLK, HIDDEN), lambda i: (i, 0)),
            pl.BlockSpec((_ROWS_BLK, HIDDEN), lambda i: (i, 0)),
        ],
        out_shape=[
            jax.ShapeDtypeStruct((N_NODES, HIDDEN), jnp.float32),
            jax.ShapeDtypeStruct((N_NODES, HIDDEN), jnp.float32),
        ],
    )(emb, ws, wd, eb)


def _node_body(x_ref, w_ref, b_ref, o_ref):
    o_ref[...] = (
        jnp.dot(x_ref[...], w_ref[...], preferred_element_type=jnp.float32,
                precision=lax.Precision.HIGHEST)
        + b_ref[...]
    )


def _node_linear(emb, w, b):
    grid = (N_NODES // _ROWS_BLK,)
    return pl.pallas_call(
        _node_body,
        grid=grid,
        in_specs=[
            pl.BlockSpec((_ROWS_BLK, HIDDEN), lambda i: (i, 0)),
            pl.BlockSpec((HIDDEN, N_NODE_FEAT), lambda i: (0, 0)),
            pl.BlockSpec((1, N_NODE_FEAT), lambda i: (0, 0)),
        ],
        out_specs=pl.BlockSpec((_ROWS_BLK, N_NODE_FEAT), lambda i: (i, 0)),
        out_shape=jax.ShapeDtypeStruct((N_NODES, N_NODE_FEAT), jnp.float32),
    )(emb, w, b)


_NC = 2
_NS = 16
_NW = _NC * _NS
_E_W = N_EDGES // _NW
_CHUNK = 128
_NFULL = _E_W // _CHUNK
_TAIL = _E_W - _NFULL * _CHUNK


def _edge_body(psrc_hbm, pdst_hbm, src_hbm, dst_hbm, out_hbm,
               src_v, dst_v,
               a0, a1, b0, b1, o0, o1,
               gs0, gs1, os0, os1):
    wid = lax.axis_index("s") * _NC + lax.axis_index("c")
    base = pl.multiple_of(wid * _E_W, 8)

    pltpu.sync_copy(src_hbm.at[pl.ds(base, _E_W)], src_v)
    pltpu.sync_copy(dst_hbm.at[pl.ds(base, _E_W)], dst_v)

    @pl.loop(0, _E_W // 16, unroll=8)
    def _(i):
        off = pl.multiple_of(i * 16, 8)
        src_v[pl.ds(off, 16)] = src_v[pl.ds(off, 16)] * 8
        dst_v[pl.ds(off, 16)] = dst_v[pl.ds(off, 16)] * 8

    abufs = (a0, a1)
    bbufs = (b0, b1)
    obufs = (o0, o1)
    gsems = (gs0, gs1)
    osems = (os0, os1)

    def issue_gather(cc, k):
        off = pl.multiple_of(cc * _CHUNK, 8)
        pltpu.async_copy(psrc_hbm.at[src_v.at[pl.ds(off, _CHUNK)]],
                         abufs[k], gsems[k])
        pltpu.async_copy(pdst_hbm.at[dst_v.at[pl.ds(off, _CHUNK)]],
                         bbufs[k], gsems[k])

    def wait_gather(k):
        pltpu.make_async_copy(psrc_hbm.at[pl.ds(0, _CHUNK)], abufs[k],
                              gsems[k]).wait()
        pltpu.make_async_copy(pdst_hbm.at[pl.ds(0, _CHUNK)], bbufs[k],
                              gsems[k]).wait()

    def compute(k):
        a_ref, b_ref, o_ref = abufs[k], bbufs[k], obufs[k]

        @pl.loop(0, _CHUNK, unroll=8)
        def _(i):
            o_ref[i] = a_ref[i] + b_ref[i]

    def issue_out(cc, k):
        off = pl.multiple_of(base + cc * _CHUNK, 8)
        pltpu.async_copy(obufs[k], out_hbm.at[pl.ds(off, _CHUNK)], osems[k])

    def wait_out(k):
        pltpu.make_async_copy(obufs[k], out_hbm.at[pl.ds(base, _CHUNK)],
                              osems[k]).wait()

    issue_gather(0, 0)
    issue_gather(1, 1)

    for cc in (0, 1):
        k = cc % 2
        wait_gather(k)
        compute(k)
        issue_out(cc, k)
        issue_gather(cc + 2, k)

    @pl.loop(2, _NFULL - 2, step=2)
    def _(c):
        for k in range(2):
            cc = c + k
            wait_gather(k)
            wait_out(k)
            compute(k)
            issue_out(cc, k)
            issue_gather(cc + 2, k)

    for cc in (_NFULL - 2, _NFULL - 1):
        k = cc % 2
        wait_gather(k)
        wait_out(k)
        compute(k)
        issue_out(cc, k)

    t_off = pl.multiple_of(_NFULL * _CHUNK, 8)
    ta = abufs[0].at[pl.ds(0, _TAIL)]
    tb = bbufs[0].at[pl.ds(0, _TAIL)]
    to = obufs[0].at[pl.ds(0, _TAIL)]
    pltpu.async_copy(psrc_hbm.at[src_v.at[pl.ds(t_off, _TAIL)]], ta, gs0)
    pltpu.async_copy(pdst_hbm.at[dst_v.at[pl.ds(t_off, _TAIL)]], tb, gs0)
    wait_out(0)
    pltpu.make_async_copy(psrc_hbm.at[pl.ds(0, _TAIL)], ta, gs0).wait()
    pltpu.make_async_copy(pdst_hbm.at[pl.ds(0, _TAIL)], tb, gs0).wait()
    for i in range(_TAIL):
        to[i] = ta[i] + tb[i]
    pltpu.async_copy(to, out_hbm.at[pl.ds(base + t_off, _TAIL)], os0)

    pltpu.make_async_copy(to, out_hbm.at[pl.ds(base, _TAIL)], os0).wait()
    wait_out(1)


def _edge_decode(psrc, pdst, src, dst):
    mesh = plsc.VectorSubcoreMesh(core_axis_name="c", subcore_axis_name="s")
    f32 = jnp.float32
    run = pl.kernel(
        _edge_body,
        out_type=jax.ShapeDtypeStruct((N_EDGES, N_EDGE_FEAT), f32),
        mesh=mesh,
        compiler_params=pltpu.CompilerParams(use_tc_tiling_on_sc=False),
        scratch_types=[
            pltpu.VMEM((_E_W,), jnp.int32),
            pltpu.VMEM((_E_W,), jnp.int32),
            pltpu.VMEM((_CHUNK, N_EDGE_FEAT), f32),
            pltpu.VMEM((_CHUNK, N_EDGE_FEAT), f32),
            pltpu.VMEM((_CHUNK, N_EDGE_FEAT), f32),
            pltpu.VMEM((_CHUNK, N_EDGE_FEAT), f32),
            pltpu.VMEM((_CHUNK, N_EDGE_FEAT), f32),
            pltpu.VMEM((_CHUNK, N_EDGE_FEAT), f32),
            pltpu.SemaphoreType.DMA,
            pltpu.SemaphoreType.DMA,
            pltpu.SemaphoreType.DMA,
            pltpu.SemaphoreType.DMA,
        ],
    )
    return run(psrc, pdst, src, dst)


def kernel(embeddings, edge_index, node_w, node_b, edge_w, edge_b):
    src = edge_index[0].astype(jnp.int32)
    dst = edge_index[1].astype(jnp.int32)

    pad = HIDDEN - N_EDGE_FEAT
    ws = jnp.pad(edge_w[:, :HIDDEN].T, ((0, 0), (0, pad)))
    wd = jnp.pad(edge_w[:, HIDDEN:].T, ((0, 0), (0, pad)))
    eb = jnp.pad(edge_b, (0, pad)).reshape(1, HIDDEN)

    psrc, pdst = _p_tables(embeddings, ws, wd, eb)
    psrc = psrc.reshape(8 * N_NODES, N_EDGE_FEAT)
    pdst = pdst.reshape(8 * N_NODES, N_EDGE_FEAT)
    edge_hat = _edge_decode(psrc, pdst, src, dst)
    node_hat = _node_linear(embeddings, node_w.T, node_b.reshape(1, N_NODE_FEAT))
    return (node_hat, edge_hat)


# speedup vs baseline: 11.2959x; 1.7991x over previous
import functools

import jax
import jax.numpy as jnp
from jax import lax
from jax.experimental import pallas as pl
from jax.experimental.pallas import tpu as pltpu
from jax.experimental.pallas import tpu_sc as plsc

HIDDEN = 128
N_NODE_FEAT = 128
N_EDGE_FEAT = 16
N_NODES = 10000
N_EDGES = 320000


_ROWS_BLK = 1000


def _p_table_body(x_ref, wc_ref, bc_ref, t_ref):
    t_ref[...] = (
        jnp.dot(x_ref[...], wc_ref[...], preferred_element_type=jnp.float32,
                precision=lax.Precision.HIGHEST)
        + bc_ref[...]
    )


def _p_table(emb, wc, bc):
    grid = (N_NODES // _ROWS_BLK,)
    return pl.pallas_call(
        _p_table_body,
        grid=grid,
        in_specs=[
            pl.BlockSpec((_ROWS_BLK, HIDDEN), lambda i: (i, 0)),
            pl.BlockSpec((HIDDEN, HIDDEN), lambda i: (0, 0)),
            pl.BlockSpec((1, HIDDEN), lambda i: (0, 0)),
        ],
        out_specs=pl.BlockSpec((_ROWS_BLK, HIDDEN), lambda i: (i, 0)),
        out_shape=jax.ShapeDtypeStruct((N_NODES, HIDDEN), jnp.float32),
    )(emb, wc, bc)


def _node_body(x_ref, w_ref, b_ref, o_ref):
    o_ref[...] = (
        jnp.dot(x_ref[...], w_ref[...], preferred_element_type=jnp.float32,
                precision=lax.Precision.HIGHEST)
        + b_ref[...]
    )


def _node_linear(emb, w, b):
    grid = (N_NODES // _ROWS_BLK,)
    return pl.pallas_call(
        _node_body,
        grid=grid,
        in_specs=[
            pl.BlockSpec((_ROWS_BLK, HIDDEN), lambda i: (i, 0)),
            pl.BlockSpec((HIDDEN, N_NODE_FEAT), lambda i: (0, 0)),
            pl.BlockSpec((1, N_NODE_FEAT), lambda i: (0, 0)),
        ],
        out_specs=pl.BlockSpec((_ROWS_BLK, N_NODE_FEAT), lambda i: (i, 0)),
        out_shape=jax.ShapeDtypeStruct((N_NODES, N_NODE_FEAT), jnp.float32),
    )(emb, w, b)


_NC = 2
_NS = 16
_NW = _NC * _NS
_BLK = 128
_NBLOCKS = N_EDGES // _BLK
_BPW = _NBLOCKS // _NW
_XTRA = _NBLOCKS - _BPW * _NW
_TILE_W = 8 * _BLK
_OUT_WORDS = N_EDGES * N_EDGE_FEAT
_HALF = _NBLOCKS * _TILE_W


def _edge_body(t_hbm, idx_hbm, out_hbm,
               slab_v, xtra_v,
               a0, a1, b0, b1, o0, o1,
               gs0, gs1, os0, os1):
    wid = lax.axis_index("s") * _NC + lax.axis_index("c")
    blk0 = wid * _BPW

    pltpu.sync_copy(idx_hbm.at[pl.ds(blk0, _BPW)], slab_v)

    cvec = lax.iota(jnp.int32, 16)
    pat = (cvec >> 3) * _TILE_W + (cvec & 7) * _BLK

    @pl.loop(0, _BPW)
    def _(j):
        for r in range(2):
            for v in range(8):
                sl = pl.ds(v * 16, 16)
                slab_v[j, r, sl] = slab_v[j, r, sl] * 8 + r

    abufs = (a0, a1)
    bbufs = (b0, b1)
    obufs = (o0, o1)
    gsems = (gs0, gs1)
    osems = (os0, os1)

    def issue_gather(j, k):
        pltpu.async_copy(t_hbm.at[slab_v.at[j, 0]], abufs[k], gsems[k])
        pltpu.async_copy(t_hbm.at[slab_v.at[j, 1]], bbufs[k], gsems[k])

    def wait_gather(k):
        pltpu.make_async_copy(t_hbm.at[pl.ds(0, _BLK)], abufs[k],
                              gsems[k]).wait()
        pltpu.make_async_copy(t_hbm.at[pl.ds(0, _BLK)], bbufs[k],
                              gsems[k]).wait()

    def compute(k):
        a_ref, b_ref, o_ref = abufs[k], bbufs[k], obufs[k]

        @pl.loop(0, _BLK, unroll=8)
        def _(i):
            plsc.store_scatter(o_ref, [pat + i], a_ref[i] + b_ref[i])

    def issue_out(j, k):
        b = blk0 + j
        o_ref = obufs[k]
        pltpu.async_copy(o_ref.at[pl.ds(0, _TILE_W)],
                         out_hbm.at[pl.ds(b * _TILE_W, _TILE_W)], osems[k])
        pltpu.async_copy(o_ref.at[pl.ds(_TILE_W, _TILE_W)],
                         out_hbm.at[pl.ds(_HALF + b * _TILE_W, _TILE_W)],
                         osems[k])

    def wait_out(k):
        pltpu.make_async_copy(obufs[k], out_hbm.at[pl.ds(0, 2 * _TILE_W)],
                              osems[k]).wait()

    issue_gather(0, 0)
    issue_gather(1, 1)

    for j in (0, 1):
        k = j % 2
        wait_gather(k)
        compute(k)
        issue_out(j, k)
        issue_gather(j + 2, k)

    @pl.loop(2, _BPW - 2, step=2)
    def _(c):
        for k in range(2):
            j = c + k
            wait_gather(k)
            wait_out(k)
            compute(k)
            issue_out(j, k)
            issue_gather(j + 2, k)

    for j in (_BPW - 2, _BPW - 1):
        k = j % 2
        wait_gather(k)
        wait_out(k)
        compute(k)
        issue_out(j, k)

    wait_out(0)
    wait_out(1)

    @pl.when(wid < _XTRA)
    def _():
        xb = _NW * _BPW + wid
        pltpu.sync_copy(idx_hbm.at[xb], xtra_v)
        for r in range(2):
            for v in range(8):
                sl = pl.ds(v * 16, 16)
                xtra_v[r, sl] = xtra_v[r, sl] * 8 + r
        pltpu.async_copy(t_hbm.at[xtra_v.at[0]], a0, gs0)
        pltpu.async_copy(t_hbm.at[xtra_v.at[1]], b0, gs0)
        wait_gather(0)

        @pl.loop(0, _BLK, unroll=8)
        def _(i):
            plsc.store_scatter(o0, [pat + i], a0[i] + b0[i])

        pltpu.async_copy(o0.at[pl.ds(0, _TILE_W)],
                         out_hbm.at[pl.ds(xb * _TILE_W, _TILE_W)], os0)
        pltpu.async_copy(o0.at[pl.ds(_TILE_W, _TILE_W)],
                         out_hbm.at[pl.ds(_HALF + xb * _TILE_W, _TILE_W)],
                         os0)
        wait_out(0)


def _edge_decode(t2, idx3):
    mesh = plsc.VectorSubcoreMesh(core_axis_name="c", subcore_axis_name="s")
    f32 = jnp.float32
    run = pl.kernel(
        _edge_body,
        out_type=jax.ShapeDtypeStruct((_OUT_WORDS,), f32),
        mesh=mesh,
        compiler_params=pltpu.CompilerParams(use_tc_tiling_on_sc=False,
                                             needs_layout_passes=False),
        scratch_types=[
            pltpu.VMEM((_BPW, 2, _BLK), jnp.int32),
            pltpu.VMEM((2, _BLK), jnp.int32),
            pltpu.VMEM((_BLK, N_EDGE_FEAT), f32),
            pltpu.VMEM((_BLK, N_EDGE_FEAT), f32),
            pltpu.VMEM((_BLK, N_EDGE_FEAT), f32),
            pltpu.VMEM((_BLK, N_EDGE_FEAT), f32),
            pltpu.VMEM((2 * _TILE_W,), f32),
            pltpu.VMEM((2 * _TILE_W,), f32),
            pltpu.SemaphoreType.DMA,
            pltpu.SemaphoreType.DMA,
            pltpu.SemaphoreType.DMA,
            pltpu.SemaphoreType.DMA,
        ],
    )
    return run(t2, idx3)


def kernel(embeddings, edge_index, node_w, node_b, edge_w, edge_b):
    pad = HIDDEN - 2 * N_EDGE_FEAT
    wc = jnp.concatenate(
        [edge_w[:, :HIDDEN].T, edge_w[:, HIDDEN:].T,
         jnp.zeros((HIDDEN, pad), jnp.float32)], axis=1)
    bc = jnp.pad(edge_b, (0, HIDDEN - N_EDGE_FEAT)).reshape(1, HIDDEN)

    t = _p_table(embeddings, wc, bc)
    t2 = t.reshape(8 * N_NODES, N_EDGE_FEAT)

    idx3 = (edge_index.astype(jnp.int32)
            .reshape(2, _NBLOCKS, _BLK)
            .transpose(1, 0, 2))

    out_flat = _edge_decode(t2, idx3)
    edge_hat = (out_flat
                .reshape(2, _NBLOCKS, 8, _BLK)
                .transpose(1, 3, 0, 2)
                .reshape(N_EDGES, N_EDGE_FEAT))

    node_hat = _node_linear(embeddings, node_w.T,
                            node_b.reshape(1, N_NODE_FEAT))
    return (node_hat, edge_hat)


# speedup vs baseline: 13.1547x; 1.1646x over previous
import functools

import jax
import jax.numpy as jnp
from jax import lax
from jax.experimental import pallas as pl
from jax.experimental.pallas import tpu as pltpu
from jax.experimental.pallas import tpu_sc as plsc

HIDDEN = 128
N_NODE_FEAT = 128
N_EDGE_FEAT = 16
N_NODES = 10000
N_EDGES = 320000


_ROWS_BLK = 1000


def _p_table_body(x_ref, wc_ref, bc_ref, t_ref):
    t_ref[...] = (
        jnp.dot(x_ref[...], wc_ref[...], preferred_element_type=jnp.float32,
                precision=lax.Precision.HIGHEST)
        + bc_ref[...]
    )


def _p_table(emb, wc, bc):
    grid = (N_NODES // _ROWS_BLK,)
    return pl.pallas_call(
        _p_table_body,
        grid=grid,
        in_specs=[
            pl.BlockSpec((_ROWS_BLK, HIDDEN), lambda i: (i, 0)),
            pl.BlockSpec((HIDDEN, HIDDEN), lambda i: (0, 0)),
            pl.BlockSpec((1, HIDDEN), lambda i: (0, 0)),
        ],
        out_specs=pl.BlockSpec((_ROWS_BLK, HIDDEN), lambda i: (i, 0)),
        out_shape=jax.ShapeDtypeStruct((N_NODES, HIDDEN), jnp.float32),
    )(emb, wc, bc)


def _node_body(x_ref, w_ref, b_ref, o_ref):
    o_ref[...] = (
        jnp.dot(x_ref[...], w_ref[...], preferred_element_type=jnp.float32,
                precision=lax.Precision.HIGHEST)
        + b_ref[...]
    )


def _node_linear(emb, w, b):
    grid = (N_NODES // _ROWS_BLK,)
    return pl.pallas_call(
        _node_body,
        grid=grid,
        in_specs=[
            pl.BlockSpec((_ROWS_BLK, HIDDEN), lambda i: (i, 0)),
            pl.BlockSpec((HIDDEN, N_NODE_FEAT), lambda i: (0, 0)),
            pl.BlockSpec((1, N_NODE_FEAT), lambda i: (0, 0)),
        ],
        out_specs=pl.BlockSpec((_ROWS_BLK, N_NODE_FEAT), lambda i: (i, 0)),
        out_shape=jax.ShapeDtypeStruct((N_NODES, N_NODE_FEAT), jnp.float32),
    )(emb, w, b)


_NC = 2
_NS = 16
_NW = _NC * _NS
_BLK = 128
_NBLOCKS = N_EDGES // _BLK
_BPW = _NBLOCKS // _NW
_XTRA = _NBLOCKS - _BPW * _NW
_TILE_W = 8 * _BLK
_OUT_WORDS = N_EDGES * N_EDGE_FEAT
_HALF = _NBLOCKS * _TILE_W


def _edge_body(t_hbm, idx_hbm, out_hbm,
               slab_v, xtra_v,
               a0, a1, b0, b1, o0, o1, s0, s1,
               gs0, gs1, os0, os1):
    wid = lax.axis_index("s") * _NC + lax.axis_index("c")
    blk0 = wid * _BPW

    pltpu.sync_copy(idx_hbm.at[pl.ds(blk0, _BPW)], slab_v)

    cvec = lax.iota(jnp.int32, 16)

    @pl.loop(0, _BPW)
    def _(j):
        for r in range(2):
            for v in range(8):
                sl = pl.ds(v * 16, 16)
                slab_v[j, r, sl] = slab_v[j, r, sl] * 8 + r

    abufs = (a0, a1)
    bbufs = (b0, b1)
    obufs = (o0, o1)
    sbufs = (s0, s1)
    gsems = (gs0, gs1)
    osems = (os0, os1)

    def issue_gather(j, k):
        pltpu.async_copy(t_hbm.at[slab_v.at[j, 0]], abufs[k], gsems[k])
        pltpu.async_copy(t_hbm.at[slab_v.at[j, 1]], bbufs[k], gsems[k])

    def wait_gather(k):
        pltpu.make_async_copy(t_hbm.at[pl.ds(0, _BLK)], abufs[k],
                              gsems[k]).wait()
        pltpu.make_async_copy(t_hbm.at[pl.ds(0, _BLK)], bbufs[k],
                              gsems[k]).wait()

    iota17 = cvec * 17

    def compute(k):
        a_ref, b_ref, o_ref, s_ref = abufs[k], bbufs[k], obufs[k], sbufs[k]

        @pl.loop(0, _BLK, unroll=8)
        def _(i):
            s_ref[pl.ds(i * 17, 16)] = a_ref[i] + b_ref[i]

        @pl.loop(0, 8)
        def _(g):
            gidx = iota17 + g * (16 * 17)
            gout = g * 16
            for c in range(16):
                v = plsc.load_gather(s_ref, [gidx + c])
                dst = (c >> 3) * _TILE_W + (c & 7) * _BLK
                o_ref[pl.ds(gout + dst, 16)] = v

    def issue_out(j, k):
        b = blk0 + j
        o_ref = obufs[k]
        pltpu.async_copy(o_ref.at[pl.ds(0, _TILE_W)],
                         out_hbm.at[pl.ds(b * _TILE_W, _TILE_W)], osems[k])
        pltpu.async_copy(o_ref.at[pl.ds(_TILE_W, _TILE_W)],
                         out_hbm.at[pl.ds(_HALF + b * _TILE_W, _TILE_W)],
                         osems[k])

    def wait_out(k):
        pltpu.make_async_copy(obufs[k], out_hbm.at[pl.ds(0, 2 * _TILE_W)],
                              osems[k]).wait()

    issue_gather(0, 0)
    issue_gather(1, 1)

    for j in (0, 1):
        k = j % 2
        wait_gather(k)
        compute(k)
        issue_out(j, k)
        issue_gather(j + 2, k)

    @pl.loop(2, _BPW - 2, step=2)
    def _(c):
        for k in range(2):
            j = c + k
            wait_gather(k)
            wait_out(k)
            compute(k)
            issue_out(j, k)
            issue_gather(j + 2, k)

    for j in (_BPW - 2, _BPW - 1):
        k = j % 2
        wait_gather(k)
        wait_out(k)
        compute(k)
        issue_out(j, k)

    wait_out(0)
    wait_out(1)

    @pl.when(wid < _XTRA)
    def _():
        xb = _NW * _BPW + wid
        pltpu.sync_copy(idx_hbm.at[xb], xtra_v)
        for r in range(2):
            for v in range(8):
                sl = pl.ds(v * 16, 16)
                xtra_v[r, sl] = xtra_v[r, sl] * 8 + r
        pltpu.async_copy(t_hbm.at[xtra_v.at[0]], a0, gs0)
        pltpu.async_copy(t_hbm.at[xtra_v.at[1]], b0, gs0)
        wait_gather(0)
        compute(0)
        pltpu.async_copy(o0.at[pl.ds(0, _TILE_W)],
                         out_hbm.at[pl.ds(xb * _TILE_W, _TILE_W)], os0)
        pltpu.async_copy(o0.at[pl.ds(_TILE_W, _TILE_W)],
                         out_hbm.at[pl.ds(_HALF + xb * _TILE_W, _TILE_W)],
                         os0)
        wait_out(0)


def _edge_decode(t2, idx3):
    mesh = plsc.VectorSubcoreMesh(core_axis_name="c", subcore_axis_name="s")
    f32 = jnp.float32
    run = pl.kernel(
        _edge_body,
        out_type=jax.ShapeDtypeStruct((_OUT_WORDS,), f32),
        mesh=mesh,
        compiler_params=pltpu.CompilerParams(use_tc_tiling_on_sc=False,
                                             needs_layout_passes=False),
        scratch_types=[
            pltpu.VMEM((_BPW, 2, _BLK), jnp.int32),
            pltpu.VMEM((2, _BLK), jnp.int32),
            pltpu.VMEM((_BLK, N_EDGE_FEAT), f32),
            pltpu.VMEM((_BLK, N_EDGE_FEAT), f32),
            pltpu.VMEM((_BLK, N_EDGE_FEAT), f32),
            pltpu.VMEM((_BLK, N_EDGE_FEAT), f32),
            pltpu.VMEM((2 * _TILE_W,), f32),
            pltpu.VMEM((2 * _TILE_W,), f32),
            pltpu.VMEM((_BLK * 17,), f32),
            pltpu.VMEM((_BLK * 17,), f32),
            pltpu.SemaphoreType.DMA,
            pltpu.SemaphoreType.DMA,
            pltpu.SemaphoreType.DMA,
            pltpu.SemaphoreType.DMA,
        ],
    )
    return run(t2, idx3)


def kernel(embeddings, edge_index, node_w, node_b, edge_w, edge_b):
    pad = HIDDEN - 2 * N_EDGE_FEAT
    wc = jnp.concatenate(
        [edge_w[:, :HIDDEN].T, edge_w[:, HIDDEN:].T,
         jnp.zeros((HIDDEN, pad), jnp.float32)], axis=1)
    bc = jnp.pad(edge_b, (0, HIDDEN - N_EDGE_FEAT)).reshape(1, HIDDEN)

    t = _p_table(embeddings, wc, bc)
    t2 = t.reshape(8 * N_NODES, N_EDGE_FEAT)

    idx3 = (edge_index.astype(jnp.int32)
            .reshape(2, _NBLOCKS, _BLK)
            .transpose(1, 0, 2))

    out_flat = _edge_decode(t2, idx3)
    edge_hat = (out_flat
                .reshape(2, _NBLOCKS, 8, _BLK)
                .transpose(1, 3, 0, 2)
                .reshape(N_EDGES, N_EDGE_FEAT))

    node_hat = _node_linear(embeddings, node_w.T,
                            node_b.reshape(1, N_NODE_FEAT))
    return (node_hat, edge_hat)
